# Initial kernel scaffold; baseline (speedup 1.0000x reference)
import functools

import jax
import jax.numpy as jnp
from jax import lax
from jax.experimental import pallas as pl
from jax.experimental.pallas import tpu as pltpu
from jax.experimental.pallas import tpu_sc as plsc

_NUM_FIELDS = 26
_VOCAB = 100000
_EMBED_DIM = 16
_BATCH = 16384

_NC = 2
_NS = 16
_L = 16
_NW = _NC * _NS

_N_ROWS = _BATCH * _NUM_FIELDS
_PER_W = _N_ROWS // _NW
_CHUNK = 1664
_N_CHUNKS = _PER_W // _CHUNK


_mesh = plsc.VectorSubcoreMesh(core_axis_name="c", subcore_axis_name="s")


@functools.partial(
    pl.kernel,
    mesh=_mesh,
    out_type=jax.ShapeDtypeStruct((_N_ROWS, _EMBED_DIM), jnp.float32),
    scratch_types=[
        pltpu.VMEM((_CHUNK,), jnp.int32),
        pltpu.VMEM((_CHUNK,), jnp.int32),
        pltpu.VMEM((_CHUNK, _EMBED_DIM), jnp.float32),
        pltpu.SemaphoreType.DMA,
    ],
)
def _gather_kernel(tab_hbm, idx_hbm, off_hbm, out_hbm, idx_v, off_v, rows_v, sem):
    wid = lax.axis_index("s") * _NC + lax.axis_index("c")
    base = wid * _PER_W
    pltpu.sync_copy(off_hbm, off_v)

    def chunk_body(c, carry):
        row0 = base + c * _CHUNK
        pltpu.sync_copy(idx_hbm.at[pl.ds(row0, _CHUNK)], idx_v)

        def add_body(i, carry2):
            sl = pl.ds(i * _L, _L)
            idx_v[sl] = idx_v[sl] + off_v[sl]
            return carry2

        lax.fori_loop(0, _CHUNK // _L, add_body, 0)
        pltpu.async_copy(tab_hbm.at[idx_v], rows_v, sem).wait()
        pltpu.sync_copy(rows_v, out_hbm.at[pl.ds(row0, _CHUNK)])
        return carry

    lax.fori_loop(0, _N_CHUNKS, chunk_body, 0)


def kernel(inputs, tables):
    flat_tab = tables.reshape(_NUM_FIELDS * _VOCAB, _EMBED_DIM)
    flat_idx = inputs.reshape(-1).astype(jnp.int32)
    offs = jnp.tile(
        jnp.arange(_NUM_FIELDS, dtype=jnp.int32) * _VOCAB, _CHUNK // _NUM_FIELDS
    )
    out = _gather_kernel(flat_tab, flat_idx, offs)
    return out.reshape(_BATCH, _NUM_FIELDS * _EMBED_DIM)


# speedup vs baseline: 1.2862x; 1.2862x over previous
import functools

import jax
import jax.numpy as jnp
from jax import lax
from jax.experimental import pallas as pl
from jax.experimental.pallas import tpu as pltpu
from jax.experimental.pallas import tpu_sc as plsc

_NUM_FIELDS = 26
_VOCAB = 100000
_EMBED_DIM = 16
_BATCH = 16384

_NC = 2
_NS = 16
_L = 16
_NW = _NC * _NS

_N_ROWS = _BATCH * _NUM_FIELDS
_PER_W = _N_ROWS // _NW
_CHUNK = 1664
_N_CHUNKS = _PER_W // _CHUNK


_mesh = plsc.VectorSubcoreMesh(core_axis_name="c", subcore_axis_name="s")


@functools.partial(
    pl.kernel,
    mesh=_mesh,
    out_type=jax.ShapeDtypeStruct((_N_ROWS, _EMBED_DIM), jnp.float32),
    scratch_types=[
        pltpu.VMEM((_CHUNK,), jnp.int32),
        pltpu.VMEM((_CHUNK,), jnp.int32),
        pltpu.VMEM((_CHUNK, _EMBED_DIM), jnp.float32),
        pltpu.SemaphoreType.DMA,
    ],
    compiler_params=pltpu.CompilerParams(use_tc_tiling_on_sc=False),
)
def _gather_kernel(tab_hbm, idx_hbm, off_hbm, out_hbm, idx_v, off_v, rows_v, sem):
    wid = lax.axis_index("s") * _NC + lax.axis_index("c")
    base = wid * _PER_W
    pltpu.sync_copy(off_hbm, off_v)

    def chunk_body(c, carry):
        row0 = base + c * _CHUNK
        pltpu.sync_copy(idx_hbm.at[pl.ds(row0, _CHUNK)], idx_v)

        def add_body(i, carry2):
            sl = pl.ds(i * _L, _L)
            idx_v[sl] = idx_v[sl] + off_v[sl]
            return carry2

        lax.fori_loop(0, _CHUNK // _L, add_body, 0)
        pltpu.async_copy(tab_hbm.at[idx_v], rows_v, sem).wait()
        pltpu.sync_copy(rows_v, out_hbm.at[pl.ds(row0, _CHUNK)])
        return carry

    lax.fori_loop(0, _N_CHUNKS, chunk_body, 0)


def kernel(inputs, tables):
    flat_tab = tables.reshape(_NUM_FIELDS * _VOCAB, _EMBED_DIM)
    flat_idx = inputs.reshape(-1).astype(jnp.int32)
    offs = jnp.tile(
        jnp.arange(_NUM_FIELDS, dtype=jnp.int32) * _VOCAB, _CHUNK // _NUM_FIELDS
    )
    out = _gather_kernel(flat_tab, flat_idx, offs)
    return out.reshape(_BATCH, _NUM_FIELDS * _EMBED_DIM)
